# baseline (speedup 1.0000x reference)
import functools
import math

import jax
import jax.numpy as jnp
from jax import lax
from jax.experimental import pallas as pl
from jax.experimental.pallas import tpu as pltpu
from jax.experimental.pallas import tpu_sc as plsc

_NC = 2
_NS = 16
_NW = _NC * _NS

_CH = 80


def _sc_gather(table, idx):
    B = idx.shape[0]
    D = table.shape[1]
    b_per_w = B // _NW
    n_ch = b_per_w // _CH
    idx3d = idx.reshape(_NW, n_ch, _CH)

    mesh = plsc.VectorSubcoreMesh(core_axis_name="c", subcore_axis_name="s")

    @functools.partial(
        pl.kernel,
        out_type=jax.ShapeDtypeStruct((B, D), table.dtype),
        mesh=mesh,
        scratch_types=[
            pltpu.VMEM((n_ch, _CH), jnp.int32),
            pltpu.VMEM((_CH, D), table.dtype),
            pltpu.SemaphoreType.DMA,
        ],
    )
    def k(table_hbm, idx_hbm, out_hbm, idx_v, rows_v, gsem):
        wid = lax.axis_index("s") * _NC + lax.axis_index("c")
        pltpu.sync_copy(idx_hbm.at[wid], idx_v)

        def body(c, carry):
            pltpu.async_copy(table_hbm.at[idx_v.at[c]], rows_v, gsem).wait()
            pltpu.sync_copy(
                rows_v, out_hbm.at[pl.ds(wid * b_per_w + c * _CH, _CH)]
            )
            return carry

        lax.fori_loop(0, n_ch, body, 0)

    return k(table, idx3d)


def _mlp_body(ni_ref, nj_ref, ef_ref, w1_ref, b1_ref, w2_ref, b2_ref, out_ref):
    x = jnp.concatenate(
        [ni_ref[...], nj_ref[...], ef_ref[...]], axis=1
    ).astype(jnp.bfloat16)
    h = lax.dot_general(
        x, w1_ref[...], (((1,), (1,)), ((), ())),
        preferred_element_type=jnp.float32,
    ) + b1_ref[...]
    h = 0.5 * h * (1.0 + lax.erf(h * (1.0 / math.sqrt(2.0))))
    out_ref[...] = lax.dot_general(
        h.astype(jnp.bfloat16), w2_ref[...], (((1,), (1,)), ((), ())),
        preferred_element_type=jnp.float32,
    ) + b2_ref[...]


def _tc_mlp(gathered, edge_feats, W1, b1, W2, b2, n_edges, e_blk):
    n_blk = n_edges // e_blk
    node_dim = gathered.shape[1]
    edge_dim = edge_feats.shape[1]
    in_dim = W1.shape[1]
    out_dim = W2.shape[0]
    return pl.pallas_call(
        _mlp_body,
        grid=(n_blk,),
        in_specs=[
            pl.BlockSpec((e_blk, node_dim), lambda e: (e, 0)),
            pl.BlockSpec((e_blk, node_dim), lambda e: (e + n_blk, 0)),
            pl.BlockSpec((e_blk, edge_dim), lambda e: (e, 0)),
            pl.BlockSpec((in_dim, in_dim), lambda e: (0, 0)),
            pl.BlockSpec((1, in_dim), lambda e: (0, 0)),
            pl.BlockSpec((out_dim, in_dim), lambda e: (0, 0)),
            pl.BlockSpec((1, out_dim), lambda e: (0, 0)),
        ],
        out_specs=pl.BlockSpec((e_blk, out_dim), lambda e: (e, 0)),
        out_shape=jax.ShapeDtypeStruct((n_edges, out_dim), jnp.float32),
    )(gathered, gathered, edge_feats, W1, b1, W2, b2)


def kernel(node_feats, edge_feats, edge_index, W1, b1, W2, b2):
    n_edges = edge_feats.shape[0]
    idx_all = edge_index.reshape(-1).astype(jnp.int32)
    gathered = _sc_gather(node_feats, idx_all)
    return _tc_mlp(
        gathered, edge_feats, W1.astype(jnp.bfloat16), b1[None, :],
        W2.astype(jnp.bfloat16), b2[None, :], n_edges, e_blk=2560,
    )


# speedup vs baseline: 1.6360x; 1.6360x over previous
import functools
import math

import jax
import jax.numpy as jnp
from jax import lax
from jax.experimental import pallas as pl
from jax.experimental.pallas import tpu as pltpu
from jax.experimental.pallas import tpu_sc as plsc

_NC = 2
_NS = 16
_NW = _NC * _NS

_CH = 80
_G = 5


def _sc_gather(table, idx):
    B = idx.shape[0]
    D = table.shape[1]
    b_per_w = B // _NW
    n_ch = b_per_w // _CH
    n_pair = n_ch // (2 * _G)
    rows_g = _G * _CH
    idx4d = idx.reshape(_NW, n_pair, 2 * _G, _CH)

    mesh = plsc.VectorSubcoreMesh(core_axis_name="c", subcore_axis_name="s")

    @functools.partial(
        pl.kernel,
        out_type=jax.ShapeDtypeStruct((B, D), table.dtype),
        mesh=mesh,
        scratch_types=[
            pltpu.VMEM((2 * _G, _CH), jnp.int32),
            pltpu.VMEM((rows_g, D), table.dtype),
            pltpu.VMEM((rows_g, D), table.dtype),
            pltpu.SemaphoreType.DMA,
            pltpu.SemaphoreType.DMA,
            pltpu.SemaphoreType.DMA,
            pltpu.SemaphoreType.DMA,
        ],
    )
    def k(table_hbm, idx_hbm, out_hbm, idx_v, buf_a, buf_b, gsem_a, gsem_b,
          ssem_a, ssem_b):
        wid = lax.axis_index("s") * _NC + lax.axis_index("c")
        base = wid * b_per_w

        def fire(half, buf, gsem):
            for b in range(_G):
                pltpu.async_copy(
                    table_hbm.at[idx_v.at[half * _G + b]],
                    buf.at[pl.ds(b * _CH, _CH)], gsem,
                )

        def drain(buf, gsem):
            for b in range(_G):
                pltpu.make_async_copy(
                    table_hbm.at[idx_v.at[0]],
                    buf.at[pl.ds(b * _CH, _CH)], gsem,
                ).wait()

        def store(s, half, buf, ssem):
            pltpu.async_copy(
                buf,
                out_hbm.at[pl.ds(base + (2 * s + half) * rows_g, rows_g)],
                ssem,
            )

        def wait_store(buf, ssem):
            pltpu.make_async_copy(
                buf, out_hbm.at[pl.ds(base, rows_g)], ssem
            ).wait()

        def supergroup(s, first):
            pltpu.sync_copy(idx_hbm.at[wid, s], idx_v)
            if not first:
                wait_store(buf_a, ssem_a)
            fire(0, buf_a, gsem_a)
            if not first:
                wait_store(buf_b, ssem_b)
            fire(1, buf_b, gsem_b)
            drain(buf_a, gsem_a)
            store(s, 0, buf_a, ssem_a)
            drain(buf_b, gsem_b)
            store(s, 1, buf_b, ssem_b)

        supergroup(0, True)
        lax.fori_loop(1, n_pair, lambda s, c: (supergroup(s, False), c)[1], 0)
        wait_store(buf_a, ssem_a)
        wait_store(buf_b, ssem_b)

    return k(table, idx4d)


def _mlp_body(ni_ref, nj_ref, eft_ref, w1_ref, b1_ref, w2_ref, b2_ref,
              out_ref):
    ef = eft_ref[...].T
    x = jnp.concatenate(
        [ni_ref[...], nj_ref[...], ef], axis=1
    ).astype(jnp.bfloat16)
    h = lax.dot_general(
        x, w1_ref[...], (((1,), (1,)), ((), ())),
        preferred_element_type=jnp.float32,
    ) + b1_ref[...]
    h = 0.5 * h * (1.0 + lax.erf(h * (1.0 / math.sqrt(2.0))))
    out_ref[...] = lax.dot_general(
        w2_ref[...], h.astype(jnp.bfloat16), (((1,), (1,)), ((), ())),
        preferred_element_type=jnp.float32,
    ) + b2_ref[...]


def _tc_mlp(gathered, ef_t, W1, b1, W2, b2, n_edges, e_blk):
    n_blk = n_edges // e_blk
    node_dim = gathered.shape[1]
    edge_dim = ef_t.shape[0]
    in_dim = W1.shape[1]
    out_dim = W2.shape[0]
    return pl.pallas_call(
        _mlp_body,
        grid=(n_blk,),
        in_specs=[
            pl.BlockSpec((e_blk, node_dim), lambda e: (e, 0)),
            pl.BlockSpec((e_blk, node_dim), lambda e: (e + n_blk, 0)),
            pl.BlockSpec((edge_dim, e_blk), lambda e: (0, e)),
            pl.BlockSpec((in_dim, in_dim), lambda e: (0, 0)),
            pl.BlockSpec((1, in_dim), lambda e: (0, 0)),
            pl.BlockSpec((out_dim, in_dim), lambda e: (0, 0)),
            pl.BlockSpec((out_dim, 1), lambda e: (0, 0)),
        ],
        out_specs=pl.BlockSpec((out_dim, e_blk), lambda e: (0, e)),
        out_shape=jax.ShapeDtypeStruct((out_dim, n_edges), jnp.float32),
    )(gathered, gathered, ef_t, W1, b1, W2, b2)


def kernel(node_feats, edge_feats, edge_index, W1, b1, W2, b2):
    n_edges = edge_feats.shape[0]
    idx_all = edge_index.reshape(-1).astype(jnp.int32)
    gathered = _sc_gather(node_feats, idx_all)
    out_t = _tc_mlp(
        gathered, edge_feats.T, W1.astype(jnp.bfloat16), b1[None, :],
        W2.astype(jnp.bfloat16), b2[:, None], n_edges, e_blk=2560,
    )
    return out_t.T


# speedup vs baseline: 1.8520x; 1.1320x over previous
import functools
import math

import jax
import jax.numpy as jnp
from jax import lax
from jax.experimental import pallas as pl
from jax.experimental.pallas import tpu as pltpu
from jax.experimental.pallas import tpu_sc as plsc

_NC = 2
_NS = 16
_NW = _NC * _NS

_CH = 80
_G = 5


def _sc_gather(table, idx):
    B = idx.shape[0]
    D = table.shape[1]
    b_per_w = B // _NW
    n_ch = b_per_w // _CH
    n_pair = n_ch // (2 * _G)
    rows_g = _G * _CH
    idx4d = idx.reshape(_NW, n_pair, 2 * _G, _CH)

    mesh = plsc.VectorSubcoreMesh(core_axis_name="c", subcore_axis_name="s")

    @functools.partial(
        pl.kernel,
        out_type=jax.ShapeDtypeStruct((B, D), table.dtype),
        mesh=mesh,
        scratch_types=[
            pltpu.VMEM((2 * _G, _CH), jnp.int32),
            pltpu.VMEM((rows_g, D), table.dtype),
            pltpu.VMEM((rows_g, D), table.dtype),
            pltpu.SemaphoreType.DMA,
            pltpu.SemaphoreType.DMA,
            pltpu.SemaphoreType.DMA,
            pltpu.SemaphoreType.DMA,
        ],
    )
    def k(table_hbm, idx_hbm, out_hbm, idx_v, buf_a, buf_b, gsem_a, gsem_b,
          ssem_a, ssem_b):
        wid = lax.axis_index("s") * _NC + lax.axis_index("c")
        base = wid * b_per_w

        def fire(half, buf, gsem):
            for b in range(_G):
                pltpu.async_copy(
                    table_hbm.at[idx_v.at[half * _G + b]],
                    buf.at[pl.ds(b * _CH, _CH)], gsem,
                )

        def drain(buf, gsem):
            for b in range(_G):
                pltpu.make_async_copy(
                    table_hbm.at[idx_v.at[0]],
                    buf.at[pl.ds(b * _CH, _CH)], gsem,
                ).wait()

        def store(s, half, buf, ssem):
            pltpu.async_copy(
                buf,
                out_hbm.at[pl.ds(base + (2 * s + half) * rows_g, rows_g)],
                ssem,
            )

        def wait_store(buf, ssem):
            pltpu.make_async_copy(
                buf, out_hbm.at[pl.ds(base, rows_g)], ssem
            ).wait()

        def supergroup(s, first):
            pltpu.sync_copy(idx_hbm.at[wid, s], idx_v)
            if not first:
                wait_store(buf_a, ssem_a)
            fire(0, buf_a, gsem_a)
            if not first:
                wait_store(buf_b, ssem_b)
            fire(1, buf_b, gsem_b)
            drain(buf_a, gsem_a)
            store(s, 0, buf_a, ssem_a)
            drain(buf_b, gsem_b)
            store(s, 1, buf_b, ssem_b)

        supergroup(0, True)
        lax.fori_loop(1, n_pair, lambda s, c: (supergroup(s, False), c)[1], 0)
        wait_store(buf_a, ssem_a)
        wait_store(buf_b, ssem_b)

    return k(table, idx4d)


def _mlp_body(prev_ref, ni_ref, nj_ref, eft_ref, w1_ref, b1_ref, w2_ref,
              b2_ref, out_ref):
    del prev_ref
    ef = eft_ref[...].T
    x = jnp.concatenate(
        [ni_ref[...], nj_ref[...], ef], axis=1
    ).astype(jnp.bfloat16)
    h = lax.dot_general(
        x, w1_ref[...], (((1,), (1,)), ((), ())),
        preferred_element_type=jnp.float32,
    ) + b1_ref[...]
    h = 0.5 * h * (1.0 + lax.erf(h * (1.0 / math.sqrt(2.0))))
    out_ref[...] = lax.dot_general(
        w2_ref[...], h.astype(jnp.bfloat16), (((1,), (1,)), ((), ())),
        preferred_element_type=jnp.float32,
    ) + b2_ref[...]


def _tc_mlp_chunk(prev, gathered, ef_t, W1, b1, W2, b2, blk0, n_cblk, e_blk):
    node_dim = gathered.shape[1]
    edge_dim = ef_t.shape[0]
    in_dim = W1.shape[1]
    out_dim = W2.shape[0]
    n_edges = ef_t.shape[1]
    first = prev is None
    if first:
        prev = ef_t
    return pl.pallas_call(
        _mlp_body,
        grid=(n_cblk,),
        in_specs=[
            pl.BlockSpec(memory_space=pl.ANY),
            pl.BlockSpec((e_blk, node_dim), lambda e: (e, 0)),
            pl.BlockSpec((e_blk, node_dim), lambda e: (e + n_cblk, 0)),
            pl.BlockSpec((edge_dim, e_blk), lambda e: (0, e + blk0)),
            pl.BlockSpec((in_dim, in_dim), lambda e: (0, 0)),
            pl.BlockSpec((1, in_dim), lambda e: (0, 0)),
            pl.BlockSpec((out_dim, in_dim), lambda e: (0, 0)),
            pl.BlockSpec((out_dim, 1), lambda e: (0, 0)),
        ],
        out_specs=pl.BlockSpec((out_dim, e_blk), lambda e: (0, e + blk0)),
        out_shape=jax.ShapeDtypeStruct((out_dim, n_edges), jnp.float32),
        input_output_aliases={} if first else {0: 0},
    )(prev, gathered, gathered, ef_t, W1, b1, W2, b2)


_N_CHUNK = 5
_E_BLK = 2560


def kernel(node_feats, edge_feats, edge_index, W1, b1, W2, b2):
    n_edges = edge_feats.shape[0]
    ec = n_edges // _N_CHUNK
    n_cblk = ec // _E_BLK
    idx = edge_index.astype(jnp.int32)
    ef_t = edge_feats.T
    w1b = W1.astype(jnp.bfloat16)
    w2b = W2.astype(jnp.bfloat16)
    b1r = b1[None, :]
    b2r = b2[:, None]
    gathered = [
        _sc_gather(
            node_feats,
            jnp.concatenate(
                [idx[0, c * ec:(c + 1) * ec], idx[1, c * ec:(c + 1) * ec]]
            ),
        )
        for c in range(_N_CHUNK)
    ]
    out_t = None
    for c in range(_N_CHUNK):
        out_t = _tc_mlp_chunk(
            out_t, gathered[c], ef_t, w1b, b1r, w2b, b2r,
            c * n_cblk, n_cblk, _E_BLK,
        )
    return out_t.T


# speedup vs baseline: 2.1109x; 1.1398x over previous
import functools
import math

import jax
import jax.numpy as jnp
from jax import lax
from jax.experimental import pallas as pl
from jax.experimental.pallas import tpu as pltpu
from jax.experimental.pallas import tpu_sc as plsc

_NC = 2
_NS = 16
_NW = _NC * _NS

_CH = 80
_G = 5


def _sc_gather(table, idx):
    N = table.shape[0]
    B = idx.shape[0]
    D = table.shape[1]
    b_per_w = B // _NW
    n_ch = b_per_w // _CH
    n_pair = n_ch // 2
    n_stage = N // _NS
    idx3d = idx.reshape(_NW, n_ch, _CH)

    mesh = plsc.VectorSubcoreMesh(core_axis_name="c", subcore_axis_name="s")

    @functools.partial(
        pl.kernel,
        out_type=jax.ShapeDtypeStruct((B, D), table.dtype),
        mesh=mesh,
        scratch_types=[
            pltpu.VMEM_SHARED((N, D), table.dtype),
            pltpu.VMEM((n_ch, _CH), jnp.int32),
            pltpu.VMEM((_CH, D), table.dtype),
            pltpu.VMEM((_CH, D), table.dtype),
            pltpu.SemaphoreType.DMA,
            pltpu.SemaphoreType.DMA,
            pltpu.SemaphoreType.DMA,
            pltpu.SemaphoreType.DMA,
        ],
    )
    def k(table_hbm, idx_hbm, out_hbm, tbl_sh, idx_v, buf_a, buf_b,
          gsem_a, gsem_b, ssem_a, ssem_b):
        cid = lax.axis_index("c")
        sid = lax.axis_index("s")
        wid = sid * _NC + cid
        base = wid * b_per_w
        pltpu.sync_copy(
            table_hbm.at[pl.ds(sid * n_stage, n_stage)],
            tbl_sh.at[pl.ds(sid * n_stage, n_stage)],
        )
        pltpu.sync_copy(idx_hbm.at[wid], idx_v)
        plsc.subcore_barrier()

        def fire(ch, buf, gsem):
            pltpu.async_copy(tbl_sh.at[idx_v.at[ch]], buf, gsem)

        def drain(buf, gsem):
            pltpu.make_async_copy(
                tbl_sh.at[idx_v.at[0]], buf, gsem
            ).wait()

        def store(ch, buf, ssem):
            pltpu.async_copy(
                buf, out_hbm.at[pl.ds(base + ch * _CH, _CH)], ssem
            )

        def wait_store(buf, ssem):
            pltpu.make_async_copy(
                buf, out_hbm.at[pl.ds(base, _CH)], ssem
            ).wait()

        def pair(p, first):
            ch = 2 * p
            if not first:
                wait_store(buf_a, ssem_a)
            fire(ch, buf_a, gsem_a)
            if not first:
                wait_store(buf_b, ssem_b)
            fire(ch + 1, buf_b, gsem_b)
            drain(buf_a, gsem_a)
            store(ch, buf_a, ssem_a)
            drain(buf_b, gsem_b)
            store(ch + 1, buf_b, ssem_b)

        pair(0, True)
        lax.fori_loop(1, n_pair, lambda p, c: (pair(p, False), c)[1], 0)
        wait_store(buf_a, ssem_a)
        wait_store(buf_b, ssem_b)

    return k(table, idx3d)


def _mlp_body(prev_ref, ni_ref, nj_ref, eft_ref, w1_ref, b1_ref, w2_ref,
              b2_ref, out_ref):
    del prev_ref
    ef = eft_ref[...].T
    x = jnp.concatenate(
        [ni_ref[...], nj_ref[...], ef], axis=1
    ).astype(jnp.bfloat16)
    h = lax.dot_general(
        x, w1_ref[...], (((1,), (1,)), ((), ())),
        preferred_element_type=jnp.float32,
    ) + b1_ref[...]
    h = 0.5 * h * (1.0 + lax.erf(h * (1.0 / math.sqrt(2.0))))
    out_ref[...] = lax.dot_general(
        w2_ref[...], h.astype(jnp.bfloat16), (((1,), (1,)), ((), ())),
        preferred_element_type=jnp.float32,
    ) + b2_ref[...]


def _tc_mlp_chunk(prev, gathered, ef_t, W1, b1, W2, b2, blk0, n_cblk, e_blk):
    node_dim = gathered.shape[1]
    edge_dim = ef_t.shape[0]
    in_dim = W1.shape[1]
    out_dim = W2.shape[0]
    n_edges = ef_t.shape[1]
    first = prev is None
    if first:
        prev = ef_t
    return pl.pallas_call(
        _mlp_body,
        grid=(n_cblk,),
        in_specs=[
            pl.BlockSpec(memory_space=pl.ANY),
            pl.BlockSpec((e_blk, node_dim), lambda e: (e, 0)),
            pl.BlockSpec((e_blk, node_dim), lambda e: (e + n_cblk, 0)),
            pl.BlockSpec((edge_dim, e_blk), lambda e: (0, e + blk0)),
            pl.BlockSpec((in_dim, in_dim), lambda e: (0, 0)),
            pl.BlockSpec((1, in_dim), lambda e: (0, 0)),
            pl.BlockSpec((out_dim, in_dim), lambda e: (0, 0)),
            pl.BlockSpec((out_dim, 1), lambda e: (0, 0)),
        ],
        out_specs=pl.BlockSpec((out_dim, e_blk), lambda e: (0, e + blk0)),
        out_shape=jax.ShapeDtypeStruct((out_dim, n_edges), jnp.float32),
        input_output_aliases={} if first else {0: 0},
    )(prev, gathered, gathered, ef_t, W1, b1, W2, b2)


_N_CHUNK = 5
_E_BLK = 2560


def kernel(node_feats, edge_feats, edge_index, W1, b1, W2, b2):
    n_edges = edge_feats.shape[0]
    ec = n_edges // _N_CHUNK
    n_cblk = ec // _E_BLK
    idx = edge_index.astype(jnp.int32)
    n_pad = (-node_feats.shape[0]) % (8 * _NS)
    node_padded = jnp.pad(node_feats, ((0, n_pad), (0, 0)))
    ef_t = edge_feats.T
    w1b = W1.astype(jnp.bfloat16)
    w2b = W2.astype(jnp.bfloat16)
    b1r = b1[None, :]
    b2r = b2[:, None]
    gathered = [
        _sc_gather(
            node_padded,
            jnp.concatenate(
                [idx[0, c * ec:(c + 1) * ec], idx[1, c * ec:(c + 1) * ec]]
            ),
        )
        for c in range(_N_CHUNK)
    ]
    out_t = None
    for c in range(_N_CHUNK):
        out_t = _tc_mlp_chunk(
            out_t, gathered[c], ef_t, w1b, b1r, w2b, b2r,
            c * n_cblk, n_cblk, _E_BLK,
        )
    return out_t.T


# speedup vs baseline: 2.1613x; 1.0238x over previous
import functools
import math

import jax
import jax.numpy as jnp
from jax import lax
from jax.experimental import pallas as pl
from jax.experimental.pallas import tpu as pltpu
from jax.experimental.pallas import tpu_sc as plsc

_NC = 2
_NS = 16
_NW = _NC * _NS

_CH = 80
_G = 5


def _sc_gather(table, idx):
    N = table.shape[0]
    B = idx.shape[0]
    D = table.shape[1]
    b_per_w = B // _NW
    n_ch = b_per_w // _CH
    n_pair = n_ch // 2
    n_stage = N // _NS
    idx3d = idx.reshape(_NW, n_ch, _CH)

    mesh = plsc.VectorSubcoreMesh(core_axis_name="c", subcore_axis_name="s")

    @functools.partial(
        pl.kernel,
        out_type=jax.ShapeDtypeStruct((B, D), table.dtype),
        mesh=mesh,
        scratch_types=[
            pltpu.VMEM_SHARED((N, D), table.dtype),
            pltpu.VMEM((n_ch, _CH), jnp.int32),
            pltpu.VMEM((_CH, D), table.dtype),
            pltpu.VMEM((_CH, D), table.dtype),
            pltpu.SemaphoreType.DMA,
            pltpu.SemaphoreType.DMA,
            pltpu.SemaphoreType.DMA,
            pltpu.SemaphoreType.DMA,
        ],
    )
    def k(table_hbm, idx_hbm, out_hbm, tbl_sh, idx_v, buf_a, buf_b,
          gsem_a, gsem_b, ssem_a, ssem_b):
        cid = lax.axis_index("c")
        sid = lax.axis_index("s")
        wid = sid * _NC + cid
        base = wid * b_per_w
        pltpu.sync_copy(
            table_hbm.at[pl.ds(sid * n_stage, n_stage)],
            tbl_sh.at[pl.ds(sid * n_stage, n_stage)],
        )
        pltpu.sync_copy(idx_hbm.at[wid], idx_v)
        plsc.subcore_barrier()

        def fire(ch, buf, gsem):
            pltpu.async_copy(tbl_sh.at[idx_v.at[ch]], buf, gsem)

        def drain(buf, gsem):
            pltpu.make_async_copy(
                tbl_sh.at[idx_v.at[0]], buf, gsem
            ).wait()

        def store(ch, buf, ssem):
            pltpu.async_copy(
                buf, out_hbm.at[pl.ds(base + ch * _CH, _CH)], ssem
            )

        def wait_store(buf, ssem):
            pltpu.make_async_copy(
                buf, out_hbm.at[pl.ds(base, _CH)], ssem
            ).wait()

        def pair(p, first):
            ch = 2 * p
            if not first:
                wait_store(buf_a, ssem_a)
            fire(ch, buf_a, gsem_a)
            if not first:
                wait_store(buf_b, ssem_b)
            fire(ch + 1, buf_b, gsem_b)
            drain(buf_a, gsem_a)
            store(ch, buf_a, ssem_a)
            drain(buf_b, gsem_b)
            store(ch + 1, buf_b, ssem_b)

        pair(0, True)
        lax.fori_loop(1, n_pair, lambda p, c: (pair(p, False), c)[1], 0)
        wait_store(buf_a, ssem_a)
        wait_store(buf_b, ssem_b)

    return k(table, idx3d)


def _mlp_body(prev_ref, ni_ref, nj_ref, eft_ref, w1_ref, b1_ref, w2_ref,
              b2_ref, out_ref):
    del prev_ref
    ef = eft_ref[...].T
    x = jnp.concatenate(
        [ni_ref[...], nj_ref[...], ef], axis=1
    ).astype(jnp.bfloat16)
    h = lax.dot_general(
        x, w1_ref[...], (((1,), (1,)), ((), ())),
        preferred_element_type=jnp.float32,
    ) + b1_ref[...]
    h = 0.5 * h * (1.0 + lax.erf(h * (1.0 / math.sqrt(2.0))))
    out_ref[...] = lax.dot_general(
        w2_ref[...], h.astype(jnp.bfloat16), (((1,), (1,)), ((), ())),
        preferred_element_type=jnp.float32,
    ) + b2_ref[...]


def _tc_mlp_chunk(prev, gathered, ef_t, W1, b1, W2, b2, blk0, n_cblk, e_blk):
    node_dim = gathered.shape[1]
    edge_dim = ef_t.shape[0]
    in_dim = W1.shape[1]
    out_dim = W2.shape[0]
    n_edges = ef_t.shape[1]
    first = prev is None
    if first:
        prev = ef_t
    return pl.pallas_call(
        _mlp_body,
        grid=(n_cblk,),
        in_specs=[
            pl.BlockSpec(memory_space=pl.ANY),
            pl.BlockSpec((e_blk, node_dim), lambda e: (e, 0)),
            pl.BlockSpec((e_blk, node_dim), lambda e: (e + n_cblk, 0)),
            pl.BlockSpec((edge_dim, e_blk), lambda e: (0, e + blk0)),
            pl.BlockSpec((in_dim, in_dim), lambda e: (0, 0)),
            pl.BlockSpec((1, in_dim), lambda e: (0, 0)),
            pl.BlockSpec((out_dim, in_dim), lambda e: (0, 0)),
            pl.BlockSpec((out_dim, 1), lambda e: (0, 0)),
        ],
        out_specs=pl.BlockSpec((out_dim, e_blk), lambda e: (0, e + blk0)),
        out_shape=jax.ShapeDtypeStruct((out_dim, n_edges), jnp.float32),
        input_output_aliases={} if first else {0: 0},
    )(prev, gathered, gathered, ef_t, W1, b1, W2, b2)


_E_BLK = 2560
_CHUNKS = (12800, 38400, 76800, 94720, 97280)


def kernel(node_feats, edge_feats, edge_index, W1, b1, W2, b2):
    n_edges = edge_feats.shape[0]
    idx = edge_index.astype(jnp.int32)
    n_pad = (-node_feats.shape[0]) % (8 * _NS)
    node_padded = jnp.pad(node_feats, ((0, n_pad), (0, 0)))
    ef_t = edge_feats.T
    w1b = W1.astype(jnp.bfloat16)
    w2b = W2.astype(jnp.bfloat16)
    b1r = b1[None, :]
    b2r = b2[:, None]
    starts = [sum(_CHUNKS[:c]) for c in range(len(_CHUNKS))]
    gathered = [
        _sc_gather(
            node_padded,
            jnp.concatenate(
                [idx[0, s:s + ec], idx[1, s:s + ec]]
            ),
        )
        for s, ec in zip(starts, _CHUNKS)
    ]
    out_t = None
    for c, (s, ec) in enumerate(zip(starts, _CHUNKS)):
        out_t = _tc_mlp_chunk(
            out_t, gathered[c], ef_t, w1b, b1r, w2b, b2r,
            s // _E_BLK, ec // _E_BLK, _E_BLK,
        )
    return out_t.T
